# baseline (speedup 1.0000x reference)
import functools

import jax
import jax.numpy as jnp
import numpy as np
from jax import lax
from jax.experimental import pallas as pl
from jax.experimental.pallas import tpu as pltpu
from jax.experimental.pallas import tpu_sc as plsc

N = 10000
E = 320000
DE = 16
H = 128
NL = 3
G = 1000
SG = 100
OUT = 64

NC = 2
NS = 16
NW = NC * NS

EPW = E // NW
ECHUNK = 40
NPAD = 10240
NPT = NPAD // NS
NZROW = 64

NP = 10240
NB = 4
GPAD = 1024
RPW = NP // NW
PCHUNK = 64
GROWS = GPAD // NS

F32 = jnp.float32


BE = 2000


def _edge_mlp_body(ea_ref, w1_ref, b1_ref, w2_ref, b2_ref, out_ref):
    t = jnp.dot(ea_ref[...].astype(jnp.bfloat16),
                w1_ref[0].astype(jnp.bfloat16), preferred_element_type=F32)
    t = jnp.maximum(t + b1_ref[0, 0], 0.0)
    e = jnp.dot(t.astype(jnp.bfloat16), w2_ref[0].astype(jnp.bfloat16),
                preferred_element_type=F32)
    out_ref[...] = e + b2_ref[0, 0]


def _edge_mlp_1(edge_attr, w1, b1, w2, b2):
    return pl.pallas_call(
        _edge_mlp_body,
        grid=(E // BE,),
        in_specs=[
            pl.BlockSpec((BE, DE), lambda i: (i, 0)),
            pl.BlockSpec((1, DE, H), lambda i: (0, 0, 0)),
            pl.BlockSpec((1, 1, H), lambda i: (0, 0, 0)),
            pl.BlockSpec((1, H, H), lambda i: (0, 0, 0)),
            pl.BlockSpec((1, 1, H), lambda i: (0, 0, 0)),
        ],
        out_specs=pl.BlockSpec((BE, H), lambda i: (i, 0)),
        out_shape=jax.ShapeDtypeStruct((E, H), F32),
    )(edge_attr, w1[None], b1.reshape(1, 1, H), w2[None],
      b2.reshape(1, 1, H))


NCH = EPW // ECHUNK


def _msg_sc(h, e_l, src, dst):
    mesh = plsc.VectorSubcoreMesh(core_axis_name="c", subcore_axis_name="s")

    @functools.partial(
        pl.kernel,
        out_type=jax.ShapeDtypeStruct((NC, NPAD, H), F32),
        mesh=mesh,
        scratch_types=[
            [pltpu.VMEM((ECHUNK, H), F32) for _ in range(3)],
            [pltpu.VMEM((ECHUNK, H), F32) for _ in range(3)],
            [pltpu.VMEM((ECHUNK,), jnp.int32) for _ in range(8)],
            [pltpu.VMEM((ECHUNK,), jnp.int32) for _ in range(8)],
            pltpu.VMEM((NZROW, H), F32),
            pltpu.VMEM_SHARED((NPAD, H), F32),
            [pltpu.SemaphoreType.DMA for _ in range(8)],
            [pltpu.SemaphoreType.DMA for _ in range(3)],
            [pltpu.SemaphoreType.DMA for _ in range(3)],
            [pltpu.SemaphoreType.DMA for _ in range(3)],
        ],
    )
    def k(h_hbm, e_hbm, src_hbm, dst_hbm, out_hbm,
          ebufs, hbufs, sidxs, didxs, zbuf, accum,
          sem_i, sem_e, sem_h, sem_s):
        c = lax.axis_index("c")
        s = lax.axis_index("s")
        base = c * (E // NC) + s * EPW

        def start_idx(kc, q):
            off = base + kc * ECHUNK
            pltpu.async_copy(src_hbm.at[pl.ds(off, ECHUNK)], sidxs[q],
                             sem_i[q])
            pltpu.async_copy(dst_hbm.at[pl.ds(off, ECHUNK)], didxs[q],
                             sem_i[q])

        def wait_idx(q):
            pltpu.make_async_copy(src_hbm.at[pl.ds(0, ECHUNK)], sidxs[q],
                                  sem_i[q]).wait()
            pltpu.make_async_copy(dst_hbm.at[pl.ds(0, ECHUNK)], didxs[q],
                                  sem_i[q]).wait()

        def start_e(kc, b):
            off = base + kc * ECHUNK
            pltpu.async_copy(e_hbm.at[pl.ds(off, ECHUNK)], ebufs[b],
                             sem_e[b])

        def start_gather(b, q):
            pltpu.async_copy(h_hbm.at[sidxs[q]], hbufs[b], sem_h[b])

        def wait_inputs(b, q):
            pltpu.make_async_copy(e_hbm.at[pl.ds(0, ECHUNK)], ebufs[b],
                                  sem_e[b]).wait()
            pltpu.make_async_copy(h_hbm.at[sidxs[q]], hbufs[b],
                                  sem_h[b]).wait()

        def compute(b):
            eb = ebufs[b]
            hb = hbufs[b]

            def erow(i, carry):
                for t in range(4):
                    row = 4 * i + t
                    for r in range(H // 16):
                        sl = pl.ds(r * 16, 16)
                        eb[row, sl] = jnp.maximum(hb[row, sl] + eb[row, sl],
                                                  0.0)
                return carry

            lax.fori_loop(0, ECHUNK // 4, erow, 0)

        def start_scatter(b, q):
            pltpu.async_copy(ebufs[b], accum.at[didxs[q]], sem_s[b],
                             add=True)

        def wait_scatter(b):
            pltpu.make_async_copy(ebufs[b], accum.at[didxs[0]],
                                  sem_s[b]).wait()

        start_idx(0, 0)
        start_idx(1, 1)
        start_idx(2, 2)
        wait_idx(0)
        start_e(0, 0)
        start_gather(0, 0)
        wait_idx(1)
        start_e(1, 1)
        start_gather(1, 1)

        zero16 = jnp.zeros((16,), F32)

        def zrow(i, carry):
            for r in range(H // 16):
                zbuf[i, pl.ds(r * 16, 16)] = zero16
            return carry

        lax.fori_loop(0, NZROW, zrow, 0)
        row0 = s * NPT
        for kk in range(NPT // NZROW):
            pltpu.sync_copy(zbuf, accum.at[pl.ds(row0 + kk * NZROW, NZROW)])
        plsc.subcore_barrier()

        def body(kc, b, q, first):
            wait_inputs(b, q)
            compute(b)
            start_scatter(b, q)

            b2 = (b + 2) % 3
            q2 = (q + 2) % 8

            @pl.when(kc + 2 < NCH)
            def _():
                if not first:
                    wait_scatter(b2)
                start_e(kc + 2, b2)
                wait_idx(q2)
                start_gather(b2, q2)

            @pl.when(kc + 3 < NCH)
            def _():
                start_idx(kc + 3, (q + 3) % 8)

        body(0, 0, 0, True)
        body(1, 1, 1, False)
        body(2, 2, 2, False)

        def unroll24(i, carry):
            for j in range(3, 27):
                kc = 24 * i + j
                body(kc, j % 3, j % 8, False)
            return carry

        nt = (NCH - 3) // 24
        lax.fori_loop(0, nt, unroll24, 0)
        for kc in range(3 + 24 * nt, NCH):
            body(kc, kc % 3, kc % 8, False)

        wait_scatter(0)
        wait_scatter(1)
        wait_scatter(2)
        plsc.subcore_barrier()
        for kk in range(NPT // NZROW):
            r0 = row0 + kk * NZROW
            pltpu.sync_copy(accum.at[pl.ds(r0, NZROW)], zbuf)
            pltpu.sync_copy(zbuf, out_hbm.at[c, pl.ds(r0, NZROW)])

    return k(h, e_l, src, dst)


def _gin_update_body(h_ref, a_ref, ep_ref, w1_ref, b1_ref, w2_ref, b2_ref,
                     g_ref, be_ref, out_ref):
    h = h_ref[...]
    a = a_ref[0, :N] + a_ref[1, :N]
    h2 = h * ep_ref[...] + a
    t = jnp.maximum(
        jnp.dot(h2, w1_ref[...], preferred_element_type=F32) + b1_ref[...], 0.0)
    h2 = jnp.dot(t, w2_ref[...], preferred_element_type=F32) + b2_ref[...]
    h2 = jnp.maximum(h2, 0.0)
    mu = jnp.mean(h2, axis=0, keepdims=True)
    xc = h2 - mu
    var = jnp.mean(xc * xc, axis=0, keepdims=True)
    out_ref[...] = xc * lax.rsqrt(var + 1e-5) * g_ref[...] + be_ref[...]


def _gin_update(h, part, ep, w1, b1, w2, b2, gamma, beta):
    return pl.pallas_call(
        _gin_update_body,
        out_shape=jax.ShapeDtypeStruct((N, H), F32),
    )(h, part, ep, w1, b1, w2, b2, gamma, beta)


def _pool_sc(rows, idx):
    mesh = plsc.VectorSubcoreMesh(core_axis_name="c", subcore_axis_name="s")

    @functools.partial(
        pl.kernel,
        out_type=jax.ShapeDtypeStruct((NC, NB, GPAD, H), F32),
        mesh=mesh,
        scratch_types=[
            pltpu.VMEM((NB, PCHUNK, H), F32),
            pltpu.VMEM((PCHUNK,), jnp.int32),
            pltpu.VMEM((GROWS, H), F32),
            [pltpu.VMEM_SHARED((GPAD, H), F32) for _ in range(NB)],
            pltpu.SemaphoreType.DMA,
        ],
    )
    def k(rows_hbm, idx_hbm, out_hbm, rbuf, ridx, zbuf, accums, sem):
        c = lax.axis_index("c")
        s = lax.axis_index("s")

        zero16 = jnp.zeros((16,), F32)

        def zrow(i, carry):
            for r in range(H // 16):
                zbuf[i, pl.ds(r * 16, 16)] = zero16
            return carry

        lax.fori_loop(0, GROWS, zrow, 0)
        off0 = s * GROWS
        for b in range(NB):
            pltpu.sync_copy(zbuf, accums[b].at[pl.ds(off0, GROWS)])
        plsc.subcore_barrier()

        base = c * (NP // NC) + s * RPW

        def chunk(kc, carry):
            off = base + kc * PCHUNK
            pltpu.sync_copy(idx_hbm.at[pl.ds(off, PCHUNK)], ridx)
            cps = [pltpu.async_copy(rows_hbm.at[b, pl.ds(off, PCHUNK)],
                                    rbuf.at[b], sem) for b in range(NB)]
            for cp in cps:
                cp.wait()
            for b in range(NB):
                pltpu.sync_copy(rbuf.at[b], accums[b].at[ridx], add=True)
            return carry

        lax.fori_loop(0, RPW // PCHUNK, chunk, 0)

        plsc.subcore_barrier()
        for b in range(NB):
            pltpu.sync_copy(accums[b].at[pl.ds(off0, GROWS)], zbuf)
            pltpu.sync_copy(zbuf, out_hbm.at[c, b, pl.ds(off0, GROWS)])

    return k(rows, idx)


def _head_body(p_ref, w_ref, sg_ref, f0w, f0b, f1w, f1b, f2w, f2b, pw, pb,
               out_ref):
    q = p_ref[0] + p_ref[1]
    hsum = jnp.concatenate([q[0, :G], q[1, :G], q[2, :G]], axis=1)
    cnt = q[3, :G, 0:1]
    gmean = hsum / jnp.maximum(cnt, 1.0)
    g = jnp.maximum(
        jnp.dot(gmean, f0w[...], preferred_element_type=F32) + f0b[...], 0.0)
    sg = sg_ref[...]
    onehot = (lax.broadcasted_iota(jnp.int32, (SG, G), 0) == sg[None, :])
    ow = onehot.astype(F32) * w_ref[...]
    norm = jnp.sum(ow, axis=1, keepdims=True)
    s2 = jnp.dot(ow, g, preferred_element_type=F32) / norm
    s2 = jnp.maximum(
        jnp.dot(s2, f1w[...], preferred_element_type=F32) + f1b[...], 0.0)
    s2 = jnp.maximum(
        jnp.dot(s2, f2w[...], preferred_element_type=F32) + f2b[...], 0.0)
    out_ref[...] = jnp.dot(s2, pw[...], preferred_element_type=F32) + pb[...]


def _head(pooled, wrow, sgb, fc0, fc1, fc2, pred):
    return pl.pallas_call(
        _head_body,
        out_shape=jax.ShapeDtypeStruct((SG, OUT), F32),
    )(pooled, wrow, sgb, fc0[0], fc0[1], fc1[0], fc1[1], fc2[0], fc2[1],
      pred[0], pred[1])


def kernel(x, edge_index, edge_attr, batch, weights, subgraph_batch, params):
    convs = params["convs"]
    bns = params["bns"]

    e_all = [_edge_mlp_1(edge_attr, cp["bw1"], cp["bb1"], cp["bw2"],
                         cp["bb2"]) for cp in convs]

    src = edge_index[0]
    dst = edge_index[1]

    h = x
    hs = []
    for l in range(NL):
        part = _msg_sc(h, e_all[l], src, dst)
        cp = convs[l]
        bp = bns[l]
        ep = (1.0 + cp["eps"]).reshape(1, 1)
        h = _gin_update(h, part, ep, cp["mw1"], cp["mb1"], cp["mw2"],
                        cp["mb2"], bp["gamma"], bp["beta"])
        hs.append(h)

    rowpad = jnp.zeros((NP - N, H), F32)
    rows4 = jnp.stack([
        jnp.concatenate([hs[0], rowpad], axis=0),
        jnp.concatenate([hs[1], rowpad], axis=0),
        jnp.concatenate([hs[2], rowpad], axis=0),
        jnp.concatenate([jnp.ones((N, H), F32), rowpad], axis=0),
    ])
    bidx = jnp.concatenate([batch, jnp.zeros((NP - N,), jnp.int32)])

    pooled = _pool_sc(rows4, bidx)

    wrow = weights.reshape(1, G)
    return _head(pooled, wrow, subgraph_batch, params["fc0"], params["fc1"],
                 params["fc2"], params["pred"])


# speedup vs baseline: 1.0166x; 1.0166x over previous
import functools

import jax
import jax.numpy as jnp
import numpy as np
from jax import lax
from jax.experimental import pallas as pl
from jax.experimental.pallas import tpu as pltpu
from jax.experimental.pallas import tpu_sc as plsc

N = 10000
E = 320000
DE = 16
H = 128
NL = 3
G = 1000
SG = 100
OUT = 64

NC = 2
NS = 16
NW = NC * NS

EPW = E // NW
ECHUNK = 40
NPAD = 10240
NPT = NPAD // NS
NZROW = 64

NP = 10240
NB = 4
GPAD = 1024
RPW = NP // NW
PCHUNK = 64
GROWS = GPAD // NS

F32 = jnp.float32


BE = 2000


def _edge_mlp_body(ea_ref, w1_ref, b1_ref, w2_ref, b2_ref, out_ref):
    t = jnp.dot(ea_ref[...].astype(jnp.bfloat16),
                w1_ref[0].astype(jnp.bfloat16), preferred_element_type=F32)
    t = jnp.maximum(t + b1_ref[0, 0], 0.0)
    e = jnp.dot(t.astype(jnp.bfloat16), w2_ref[0].astype(jnp.bfloat16),
                preferred_element_type=F32)
    out_ref[...] = e + b2_ref[0, 0]


def _edge_mlp_1(edge_attr, w1, b1, w2, b2):
    return pl.pallas_call(
        _edge_mlp_body,
        grid=(E // BE,),
        in_specs=[
            pl.BlockSpec((BE, DE), lambda i: (i, 0)),
            pl.BlockSpec((1, DE, H), lambda i: (0, 0, 0)),
            pl.BlockSpec((1, 1, H), lambda i: (0, 0, 0)),
            pl.BlockSpec((1, H, H), lambda i: (0, 0, 0)),
            pl.BlockSpec((1, 1, H), lambda i: (0, 0, 0)),
        ],
        out_specs=pl.BlockSpec((BE, H), lambda i: (i, 0)),
        out_shape=jax.ShapeDtypeStruct((E, H), F32),
    )(edge_attr, w1[None], b1.reshape(1, 1, H), w2[None],
      b2.reshape(1, 1, H))


NCH = EPW // ECHUNK


def _msg_sc(h, e_l, src, dst):
    mesh = plsc.VectorSubcoreMesh(core_axis_name="c", subcore_axis_name="s")

    @functools.partial(
        pl.kernel,
        out_type=jax.ShapeDtypeStruct((NC, NPAD, H), F32),
        mesh=mesh,
        scratch_types=[
            [pltpu.VMEM((ECHUNK, H), F32) for _ in range(3)],
            [pltpu.VMEM((ECHUNK, H), F32) for _ in range(3)],
            [pltpu.VMEM((ECHUNK,), jnp.int32) for _ in range(4)],
            [pltpu.VMEM((ECHUNK,), jnp.int32) for _ in range(4)],
            pltpu.VMEM((NZROW, H), F32),
            pltpu.VMEM_SHARED((NPAD, H), F32),
            [pltpu.SemaphoreType.DMA for _ in range(4)],
            [pltpu.SemaphoreType.DMA for _ in range(3)],
            [pltpu.SemaphoreType.DMA for _ in range(3)],
            [pltpu.SemaphoreType.DMA for _ in range(3)],
        ],
    )
    def k(h_hbm, e_hbm, src_hbm, dst_hbm, out_hbm,
          ebufs, hbufs, sidxs, didxs, zbuf, accum,
          sem_i, sem_e, sem_h, sem_s):
        c = lax.axis_index("c")
        s = lax.axis_index("s")
        base = c * (E // NC) + s * EPW

        def start_idx(kc, q):
            off = base + kc * ECHUNK
            pltpu.async_copy(src_hbm.at[pl.ds(off, ECHUNK)], sidxs[q],
                             sem_i[q])
            pltpu.async_copy(dst_hbm.at[pl.ds(off, ECHUNK)], didxs[q],
                             sem_i[q])

        def wait_idx(q):
            pltpu.make_async_copy(src_hbm.at[pl.ds(0, ECHUNK)], sidxs[q],
                                  sem_i[q]).wait()
            pltpu.make_async_copy(dst_hbm.at[pl.ds(0, ECHUNK)], didxs[q],
                                  sem_i[q]).wait()

        def start_e(kc, b):
            off = base + kc * ECHUNK
            pltpu.async_copy(e_hbm.at[pl.ds(off, ECHUNK)], ebufs[b],
                             sem_e[b])

        def start_gather(b, q):
            pltpu.async_copy(h_hbm.at[sidxs[q]], hbufs[b], sem_h[b])

        def wait_inputs(b, q):
            pltpu.make_async_copy(e_hbm.at[pl.ds(0, ECHUNK)], ebufs[b],
                                  sem_e[b]).wait()
            pltpu.make_async_copy(h_hbm.at[sidxs[q]], hbufs[b],
                                  sem_h[b]).wait()

        def compute(b):
            eb = ebufs[b]
            hb = hbufs[b]

            def erow(i, carry):
                for r in range(H // 16):
                    sl = pl.ds(r * 16, 16)
                    eb[i, sl] = jnp.maximum(hb[i, sl] + eb[i, sl], 0.0)
                return carry

            lax.fori_loop(0, ECHUNK, erow, 0)

        def start_scatter(b, q):
            pltpu.async_copy(ebufs[b], accum.at[didxs[q]], sem_s[b],
                             add=True)

        def wait_scatter(b):
            pltpu.make_async_copy(ebufs[b], accum.at[didxs[0]],
                                  sem_s[b]).wait()

        start_idx(0, 0)
        start_idx(1, 1)
        start_idx(2, 2)
        wait_idx(0)
        start_e(0, 0)
        start_gather(0, 0)
        wait_idx(1)
        start_e(1, 1)
        start_gather(1, 1)

        zero16 = jnp.zeros((16,), F32)

        def zrow(i, carry):
            for r in range(H // 16):
                zbuf[i, pl.ds(r * 16, 16)] = zero16
            return carry

        lax.fori_loop(0, NZROW, zrow, 0)
        row0 = s * NPT
        for kk in range(NPT // NZROW):
            pltpu.sync_copy(zbuf, accum.at[pl.ds(row0 + kk * NZROW, NZROW)])
        plsc.subcore_barrier()

        def body(kc, b, q, first):
            wait_inputs(b, q)
            compute(b)
            start_scatter(b, q)

            b2 = (b + 2) % 3
            q2 = (q + 2) % 4

            @pl.when(kc + 2 < NCH)
            def _():
                if not first:
                    wait_scatter(b2)
                start_e(kc + 2, b2)
                wait_idx(q2)
                start_gather(b2, q2)

            @pl.when(kc + 3 < NCH)
            def _():
                start_idx(kc + 3, (q + 3) % 4)

        body(0, 0, 0, True)

        def twelve(i, carry):
            for j in range(1, 13):
                kc = 12 * i + j
                body(kc, j % 3, j % 4, False)
            return carry

        nt = (NCH - 1) // 12
        lax.fori_loop(0, nt, twelve, 0)
        for kc in range(1 + 12 * nt, NCH):
            body(kc, kc % 3, kc % 4, False)

        wait_scatter(0)
        wait_scatter(1)
        wait_scatter(2)
        plsc.subcore_barrier()
        for kk in range(NPT // NZROW):
            r0 = row0 + kk * NZROW
            pltpu.sync_copy(accum.at[pl.ds(r0, NZROW)], zbuf)
            pltpu.sync_copy(zbuf, out_hbm.at[c, pl.ds(r0, NZROW)])

    return k(h, e_l, src, dst)


def _gin_update_body(h_ref, a_ref, ep_ref, w1_ref, b1_ref, w2_ref, b2_ref,
                     g_ref, be_ref, out_ref):
    h = h_ref[...]
    a = a_ref[0, :N] + a_ref[1, :N]
    h2 = h * ep_ref[...] + a
    t = jnp.maximum(
        jnp.dot(h2, w1_ref[...], preferred_element_type=F32) + b1_ref[...], 0.0)
    h2 = jnp.dot(t, w2_ref[...], preferred_element_type=F32) + b2_ref[...]
    h2 = jnp.maximum(h2, 0.0)
    mu = jnp.mean(h2, axis=0, keepdims=True)
    xc = h2 - mu
    var = jnp.mean(xc * xc, axis=0, keepdims=True)
    out_ref[...] = xc * lax.rsqrt(var + 1e-5) * g_ref[...] + be_ref[...]


def _gin_update(h, part, ep, w1, b1, w2, b2, gamma, beta):
    return pl.pallas_call(
        _gin_update_body,
        out_shape=jax.ShapeDtypeStruct((N, H), F32),
    )(h, part, ep, w1, b1, w2, b2, gamma, beta)


def _pool_sc(rows, idx):
    mesh = plsc.VectorSubcoreMesh(core_axis_name="c", subcore_axis_name="s")

    @functools.partial(
        pl.kernel,
        out_type=jax.ShapeDtypeStruct((NC, NB, GPAD, H), F32),
        mesh=mesh,
        scratch_types=[
            pltpu.VMEM((NB, PCHUNK, H), F32),
            pltpu.VMEM((PCHUNK,), jnp.int32),
            pltpu.VMEM((GROWS, H), F32),
            [pltpu.VMEM_SHARED((GPAD, H), F32) for _ in range(NB)],
            pltpu.SemaphoreType.DMA,
        ],
    )
    def k(rows_hbm, idx_hbm, out_hbm, rbuf, ridx, zbuf, accums, sem):
        c = lax.axis_index("c")
        s = lax.axis_index("s")

        zero16 = jnp.zeros((16,), F32)

        def zrow(i, carry):
            for r in range(H // 16):
                zbuf[i, pl.ds(r * 16, 16)] = zero16
            return carry

        lax.fori_loop(0, GROWS, zrow, 0)
        off0 = s * GROWS
        for b in range(NB):
            pltpu.sync_copy(zbuf, accums[b].at[pl.ds(off0, GROWS)])
        plsc.subcore_barrier()

        base = c * (NP // NC) + s * RPW

        def chunk(kc, carry):
            off = base + kc * PCHUNK
            pltpu.sync_copy(idx_hbm.at[pl.ds(off, PCHUNK)], ridx)
            cps = [pltpu.async_copy(rows_hbm.at[b, pl.ds(off, PCHUNK)],
                                    rbuf.at[b], sem) for b in range(NB)]
            for cp in cps:
                cp.wait()
            for b in range(NB):
                pltpu.sync_copy(rbuf.at[b], accums[b].at[ridx], add=True)
            return carry

        lax.fori_loop(0, RPW // PCHUNK, chunk, 0)

        plsc.subcore_barrier()
        for b in range(NB):
            pltpu.sync_copy(accums[b].at[pl.ds(off0, GROWS)], zbuf)
            pltpu.sync_copy(zbuf, out_hbm.at[c, b, pl.ds(off0, GROWS)])

    return k(rows, idx)


def _head_body(p_ref, w_ref, sg_ref, f0w, f0b, f1w, f1b, f2w, f2b, pw, pb,
               out_ref):
    q = p_ref[0] + p_ref[1]
    hsum = jnp.concatenate([q[0, :G], q[1, :G], q[2, :G]], axis=1)
    cnt = q[3, :G, 0:1]
    gmean = hsum / jnp.maximum(cnt, 1.0)
    g = jnp.maximum(
        jnp.dot(gmean, f0w[...], preferred_element_type=F32) + f0b[...], 0.0)
    sg = sg_ref[...]
    onehot = (lax.broadcasted_iota(jnp.int32, (SG, G), 0) == sg[None, :])
    ow = onehot.astype(F32) * w_ref[...]
    norm = jnp.sum(ow, axis=1, keepdims=True)
    s2 = jnp.dot(ow, g, preferred_element_type=F32) / norm
    s2 = jnp.maximum(
        jnp.dot(s2, f1w[...], preferred_element_type=F32) + f1b[...], 0.0)
    s2 = jnp.maximum(
        jnp.dot(s2, f2w[...], preferred_element_type=F32) + f2b[...], 0.0)
    out_ref[...] = jnp.dot(s2, pw[...], preferred_element_type=F32) + pb[...]


def _head(pooled, wrow, sgb, fc0, fc1, fc2, pred):
    return pl.pallas_call(
        _head_body,
        out_shape=jax.ShapeDtypeStruct((SG, OUT), F32),
    )(pooled, wrow, sgb, fc0[0], fc0[1], fc1[0], fc1[1], fc2[0], fc2[1],
      pred[0], pred[1])


def kernel(x, edge_index, edge_attr, batch, weights, subgraph_batch, params):
    convs = params["convs"]
    bns = params["bns"]

    e_all = [_edge_mlp_1(edge_attr, cp["bw1"], cp["bb1"], cp["bw2"],
                         cp["bb2"]) for cp in convs]

    src = edge_index[0]
    dst = edge_index[1]

    h = x
    hs = []
    for l in range(NL):
        part = _msg_sc(h, e_all[l], src, dst)
        cp = convs[l]
        bp = bns[l]
        ep = (1.0 + cp["eps"]).reshape(1, 1)
        h = _gin_update(h, part, ep, cp["mw1"], cp["mb1"], cp["mw2"],
                        cp["mb2"], bp["gamma"], bp["beta"])
        hs.append(h)

    rowpad = jnp.zeros((NP - N, H), F32)
    rows4 = jnp.stack([
        jnp.concatenate([hs[0], rowpad], axis=0),
        jnp.concatenate([hs[1], rowpad], axis=0),
        jnp.concatenate([hs[2], rowpad], axis=0),
        jnp.concatenate([jnp.ones((N, H), F32), rowpad], axis=0),
    ])
    bidx = jnp.concatenate([batch, jnp.zeros((NP - N,), jnp.int32)])

    pooled = _pool_sc(rows4, bidx)

    wrow = weights.reshape(1, G)
    return _head(pooled, wrow, subgraph_batch, params["fc0"], params["fc1"],
                 params["fc2"], params["pred"])


# speedup vs baseline: 1.0180x; 1.0013x over previous
import functools

import jax
import jax.numpy as jnp
import numpy as np
from jax import lax
from jax.experimental import pallas as pl
from jax.experimental.pallas import tpu as pltpu
from jax.experimental.pallas import tpu_sc as plsc

N = 10000
E = 320000
DE = 16
H = 128
NL = 3
G = 1000
SG = 100
OUT = 64

NC = 2
NS = 16
NW = NC * NS

EPW = E // NW
ECHUNK = 40
NPAD = 10240
NPT = NPAD // NS
NZROW = 64

NP = 10240
NB = 4
GPAD = 1024
RPW = NP // NW
PCHUNK = 64
GROWS = GPAD // NS

F32 = jnp.float32


BE = 2000


def _edge_mlp_body(ea_ref, w1_ref, b1_ref, w2_ref, b2_ref, out_ref):
    t = jnp.dot(ea_ref[...].astype(jnp.bfloat16),
                w1_ref[0].astype(jnp.bfloat16), preferred_element_type=F32)
    t = jnp.maximum(t + b1_ref[0, 0], 0.0)
    e = jnp.dot(t.astype(jnp.bfloat16), w2_ref[0].astype(jnp.bfloat16),
                preferred_element_type=F32)
    out_ref[...] = e + b2_ref[0, 0]


def _edge_mlp_1(edge_attr, w1, b1, w2, b2):
    return pl.pallas_call(
        _edge_mlp_body,
        grid=(E // BE,),
        in_specs=[
            pl.BlockSpec((BE, DE), lambda i: (i, 0)),
            pl.BlockSpec((1, DE, H), lambda i: (0, 0, 0)),
            pl.BlockSpec((1, 1, H), lambda i: (0, 0, 0)),
            pl.BlockSpec((1, H, H), lambda i: (0, 0, 0)),
            pl.BlockSpec((1, 1, H), lambda i: (0, 0, 0)),
        ],
        out_specs=pl.BlockSpec((BE, H), lambda i: (i, 0)),
        out_shape=jax.ShapeDtypeStruct((E, H), F32),
    )(edge_attr, w1[None], b1.reshape(1, 1, H), w2[None],
      b2.reshape(1, 1, H))


NCH = EPW // ECHUNK


def _msg_sc(h, e_l, src, dst):
    mesh = plsc.VectorSubcoreMesh(core_axis_name="c", subcore_axis_name="s")

    @functools.partial(
        pl.kernel,
        out_type=jax.ShapeDtypeStruct((NC, NPAD, H), F32),
        mesh=mesh,
        scratch_types=[
            [pltpu.VMEM((ECHUNK, H), F32) for _ in range(3)],
            [pltpu.VMEM((ECHUNK, H), F32) for _ in range(3)],
            [pltpu.VMEM((ECHUNK,), jnp.int32) for _ in range(4)],
            [pltpu.VMEM((ECHUNK,), jnp.int32) for _ in range(4)],
            pltpu.VMEM((NZROW, H), F32),
            pltpu.VMEM_SHARED((NPAD, H), F32),
            [pltpu.SemaphoreType.DMA for _ in range(4)],
            [pltpu.SemaphoreType.DMA for _ in range(3)],
            [pltpu.SemaphoreType.DMA for _ in range(3)],
            [pltpu.SemaphoreType.DMA for _ in range(3)],
        ],
    )
    def k(h_hbm, e_hbm, src_hbm, dst_hbm, out_hbm,
          ebufs, hbufs, sidxs, didxs, zbuf, accum,
          sem_i, sem_e, sem_h, sem_s):
        c = lax.axis_index("c")
        s = lax.axis_index("s")
        base = c * (E // NC) + s * EPW

        def start_idx(kc, q):
            off = base + kc * ECHUNK
            pltpu.async_copy(src_hbm.at[pl.ds(off, ECHUNK)], sidxs[q],
                             sem_i[q])
            pltpu.async_copy(dst_hbm.at[pl.ds(off, ECHUNK)], didxs[q],
                             sem_i[q])

        def wait_idx(q):
            pltpu.make_async_copy(src_hbm.at[pl.ds(0, ECHUNK)], sidxs[q],
                                  sem_i[q]).wait()
            pltpu.make_async_copy(dst_hbm.at[pl.ds(0, ECHUNK)], didxs[q],
                                  sem_i[q]).wait()

        def start_e(kc, b):
            off = base + kc * ECHUNK
            pltpu.async_copy(e_hbm.at[pl.ds(off, ECHUNK)], ebufs[b],
                             sem_e[b])

        def start_gather(b, q):
            pltpu.async_copy(h_hbm.at[sidxs[q]], hbufs[b], sem_h[b])

        def wait_inputs(b, q):
            pltpu.make_async_copy(e_hbm.at[pl.ds(0, ECHUNK)], ebufs[b],
                                  sem_e[b]).wait()
            pltpu.make_async_copy(h_hbm.at[sidxs[q]], hbufs[b],
                                  sem_h[b]).wait()

        def compute(b):
            eb = ebufs[b]
            hb = hbufs[b]

            def erow(i, carry):
                for r in range(H // 16):
                    sl = pl.ds(r * 16, 16)
                    eb[i, sl] = jnp.maximum(hb[i, sl] + eb[i, sl], 0.0)
                return carry

            lax.fori_loop(0, ECHUNK, erow, 0)

        def start_scatter(b, q):
            pltpu.async_copy(ebufs[b], accum.at[didxs[q]], sem_s[b],
                             add=True)

        def wait_scatter(b):
            pltpu.make_async_copy(ebufs[b], accum.at[didxs[0]],
                                  sem_s[b]).wait()

        start_idx(0, 0)
        start_idx(1, 1)
        start_idx(2, 2)
        wait_idx(0)
        start_e(0, 0)
        start_gather(0, 0)
        wait_idx(1)
        start_e(1, 1)
        start_gather(1, 1)

        zero16 = jnp.zeros((16,), F32)

        def zrow(i, carry):
            for r in range(H // 16):
                zbuf[i, pl.ds(r * 16, 16)] = zero16
            return carry

        lax.fori_loop(0, NZROW, zrow, 0)
        row0 = s * NPT
        zcps = [pltpu.async_copy(
            zbuf, accum.at[pl.ds(row0 + kk * NZROW, NZROW)], sem_s[0])
            for kk in range(NPT // NZROW)]
        for cp in zcps:
            cp.wait()
        plsc.subcore_barrier()

        def body(kc, b, q, first):
            wait_inputs(b, q)
            compute(b)
            start_scatter(b, q)

            b2 = (b + 2) % 3
            q2 = (q + 2) % 4

            @pl.when(kc + 2 < NCH)
            def _():
                if not first:
                    wait_scatter(b2)
                start_e(kc + 2, b2)
                wait_idx(q2)
                start_gather(b2, q2)

            @pl.when(kc + 3 < NCH)
            def _():
                start_idx(kc + 3, (q + 3) % 4)

        body(0, 0, 0, True)

        def twelve(i, carry):
            for j in range(1, 13):
                kc = 12 * i + j
                body(kc, j % 3, j % 4, False)
            return carry

        nt = (NCH - 1) // 12
        lax.fori_loop(0, nt, twelve, 0)
        for kc in range(1 + 12 * nt, NCH):
            body(kc, kc % 3, kc % 4, False)

        wait_scatter(0)
        wait_scatter(1)
        wait_scatter(2)
        plsc.subcore_barrier()
        for kk in range(NPT // NZROW):
            r0 = row0 + kk * NZROW
            pltpu.sync_copy(accum.at[pl.ds(r0, NZROW)], zbuf)
            pltpu.sync_copy(zbuf, out_hbm.at[c, pl.ds(r0, NZROW)])

    return k(h, e_l, src, dst)


def _gin_update_body(h_ref, a_ref, ep_ref, w1_ref, b1_ref, w2_ref, b2_ref,
                     g_ref, be_ref, out_ref):
    h = h_ref[...]
    a = a_ref[0, :N] + a_ref[1, :N]
    h2 = h * ep_ref[...] + a
    t = jnp.maximum(
        jnp.dot(h2, w1_ref[...], preferred_element_type=F32) + b1_ref[...], 0.0)
    h2 = jnp.dot(t, w2_ref[...], preferred_element_type=F32) + b2_ref[...]
    h2 = jnp.maximum(h2, 0.0)
    mu = jnp.mean(h2, axis=0, keepdims=True)
    xc = h2 - mu
    var = jnp.mean(xc * xc, axis=0, keepdims=True)
    out_ref[...] = xc * lax.rsqrt(var + 1e-5) * g_ref[...] + be_ref[...]


def _gin_update(h, part, ep, w1, b1, w2, b2, gamma, beta):
    return pl.pallas_call(
        _gin_update_body,
        out_shape=jax.ShapeDtypeStruct((N, H), F32),
    )(h, part, ep, w1, b1, w2, b2, gamma, beta)


def _pool_sc(rows, idx):
    mesh = plsc.VectorSubcoreMesh(core_axis_name="c", subcore_axis_name="s")

    @functools.partial(
        pl.kernel,
        out_type=jax.ShapeDtypeStruct((NC, NB, GPAD, H), F32),
        mesh=mesh,
        scratch_types=[
            pltpu.VMEM((NB, PCHUNK, H), F32),
            pltpu.VMEM((PCHUNK,), jnp.int32),
            pltpu.VMEM((GROWS, H), F32),
            [pltpu.VMEM_SHARED((GPAD, H), F32) for _ in range(NB)],
            pltpu.SemaphoreType.DMA,
        ],
    )
    def k(rows_hbm, idx_hbm, out_hbm, rbuf, ridx, zbuf, accums, sem):
        c = lax.axis_index("c")
        s = lax.axis_index("s")

        zero16 = jnp.zeros((16,), F32)

        def zrow(i, carry):
            for r in range(H // 16):
                zbuf[i, pl.ds(r * 16, 16)] = zero16
            return carry

        lax.fori_loop(0, GROWS, zrow, 0)
        off0 = s * GROWS
        for b in range(NB):
            pltpu.sync_copy(zbuf, accums[b].at[pl.ds(off0, GROWS)])
        plsc.subcore_barrier()

        base = c * (NP // NC) + s * RPW

        def chunk(kc, carry):
            off = base + kc * PCHUNK
            pltpu.sync_copy(idx_hbm.at[pl.ds(off, PCHUNK)], ridx)
            cps = [pltpu.async_copy(rows_hbm.at[b, pl.ds(off, PCHUNK)],
                                    rbuf.at[b], sem) for b in range(NB)]
            for cp in cps:
                cp.wait()
            for b in range(NB):
                pltpu.sync_copy(rbuf.at[b], accums[b].at[ridx], add=True)
            return carry

        lax.fori_loop(0, RPW // PCHUNK, chunk, 0)

        plsc.subcore_barrier()
        for b in range(NB):
            pltpu.sync_copy(accums[b].at[pl.ds(off0, GROWS)], zbuf)
            pltpu.sync_copy(zbuf, out_hbm.at[c, b, pl.ds(off0, GROWS)])

    return k(rows, idx)


def _head_body(p_ref, w_ref, sg_ref, f0w, f0b, f1w, f1b, f2w, f2b, pw, pb,
               out_ref):
    q = p_ref[0] + p_ref[1]
    hsum = jnp.concatenate([q[0, :G], q[1, :G], q[2, :G]], axis=1)
    cnt = q[3, :G, 0:1]
    gmean = hsum / jnp.maximum(cnt, 1.0)
    g = jnp.maximum(
        jnp.dot(gmean, f0w[...], preferred_element_type=F32) + f0b[...], 0.0)
    sg = sg_ref[...]
    onehot = (lax.broadcasted_iota(jnp.int32, (SG, G), 0) == sg[None, :])
    ow = onehot.astype(F32) * w_ref[...]
    norm = jnp.sum(ow, axis=1, keepdims=True)
    s2 = jnp.dot(ow, g, preferred_element_type=F32) / norm
    s2 = jnp.maximum(
        jnp.dot(s2, f1w[...], preferred_element_type=F32) + f1b[...], 0.0)
    s2 = jnp.maximum(
        jnp.dot(s2, f2w[...], preferred_element_type=F32) + f2b[...], 0.0)
    out_ref[...] = jnp.dot(s2, pw[...], preferred_element_type=F32) + pb[...]


def _head(pooled, wrow, sgb, fc0, fc1, fc2, pred):
    return pl.pallas_call(
        _head_body,
        out_shape=jax.ShapeDtypeStruct((SG, OUT), F32),
    )(pooled, wrow, sgb, fc0[0], fc0[1], fc1[0], fc1[1], fc2[0], fc2[1],
      pred[0], pred[1])


def kernel(x, edge_index, edge_attr, batch, weights, subgraph_batch, params):
    convs = params["convs"]
    bns = params["bns"]

    e_all = [_edge_mlp_1(edge_attr, cp["bw1"], cp["bb1"], cp["bw2"],
                         cp["bb2"]) for cp in convs]

    src = edge_index[0]
    dst = edge_index[1]

    h = x
    hs = []
    for l in range(NL):
        part = _msg_sc(h, e_all[l], src, dst)
        cp = convs[l]
        bp = bns[l]
        ep = (1.0 + cp["eps"]).reshape(1, 1)
        h = _gin_update(h, part, ep, cp["mw1"], cp["mb1"], cp["mw2"],
                        cp["mb2"], bp["gamma"], bp["beta"])
        hs.append(h)

    rowpad = jnp.zeros((NP - N, H), F32)
    rows4 = jnp.stack([
        jnp.concatenate([hs[0], rowpad], axis=0),
        jnp.concatenate([hs[1], rowpad], axis=0),
        jnp.concatenate([hs[2], rowpad], axis=0),
        jnp.concatenate([jnp.ones((N, H), F32), rowpad], axis=0),
    ])
    bidx = jnp.concatenate([batch, jnp.zeros((NP - N,), jnp.int32)])

    pooled = _pool_sc(rows4, bidx)

    wrow = weights.reshape(1, G)
    return _head(pooled, wrow, subgraph_batch, params["fc0"], params["fc1"],
                 params["fc2"], params["pred"])


# speedup vs baseline: 1.0460x; 1.0275x over previous
import functools

import jax
import jax.numpy as jnp
import numpy as np
from jax import lax
from jax.experimental import pallas as pl
from jax.experimental.pallas import tpu as pltpu
from jax.experimental.pallas import tpu_sc as plsc

N = 10000
E = 320000
DE = 16
H = 128
NL = 3
G = 1000
SG = 100
OUT = 64

NC = 2
NS = 16
NW = NC * NS

EPW = E // NW
ECHUNK = 40
NPAD = 10240
NPT = NPAD // NS
NZROW = 64

NP = 10240
NB = 4
GPAD = 1024
RPW = NP // NW
PCHUNK = 64
GROWS = GPAD // NS

F32 = jnp.float32


BE = 4000


def _edge_mlp_body(ea_ref, w1_ref, b1_ref, w2_ref, b2_ref, out_ref):
    t = jnp.dot(ea_ref[...].astype(jnp.bfloat16),
                w1_ref[0].astype(jnp.bfloat16), preferred_element_type=F32)
    t = jnp.maximum(t + b1_ref[0, 0], 0.0)
    e = jnp.dot(t.astype(jnp.bfloat16), w2_ref[0].astype(jnp.bfloat16),
                preferred_element_type=F32)
    out_ref[...] = e + b2_ref[0, 0]


def _edge_mlp_1(edge_attr, w1, b1, w2, b2):
    return pl.pallas_call(
        _edge_mlp_body,
        grid=(E // BE,),
        in_specs=[
            pl.BlockSpec((BE, DE), lambda i: (i, 0)),
            pl.BlockSpec((1, DE, H), lambda i: (0, 0, 0)),
            pl.BlockSpec((1, 1, H), lambda i: (0, 0, 0)),
            pl.BlockSpec((1, H, H), lambda i: (0, 0, 0)),
            pl.BlockSpec((1, 1, H), lambda i: (0, 0, 0)),
        ],
        out_specs=pl.BlockSpec((BE, H), lambda i: (i, 0)),
        out_shape=jax.ShapeDtypeStruct((E, H), F32),
    )(edge_attr, w1[None], b1.reshape(1, 1, H), w2[None],
      b2.reshape(1, 1, H))


NCH = EPW // ECHUNK


def _msg_sc(h, e_l, src, dst):
    mesh = plsc.VectorSubcoreMesh(core_axis_name="c", subcore_axis_name="s")

    @functools.partial(
        pl.kernel,
        out_type=jax.ShapeDtypeStruct((NC, NPAD, H), F32),
        mesh=mesh,
        scratch_types=[
            [pltpu.VMEM((ECHUNK, H), F32) for _ in range(3)],
            [pltpu.VMEM((ECHUNK, H), F32) for _ in range(3)],
            [pltpu.VMEM((ECHUNK,), jnp.int32) for _ in range(4)],
            [pltpu.VMEM((ECHUNK,), jnp.int32) for _ in range(4)],
            pltpu.VMEM((NZROW, H), F32),
            pltpu.VMEM_SHARED((NPAD, H), F32),
            [pltpu.SemaphoreType.DMA for _ in range(4)],
            [pltpu.SemaphoreType.DMA for _ in range(3)],
            [pltpu.SemaphoreType.DMA for _ in range(3)],
            [pltpu.SemaphoreType.DMA for _ in range(3)],
        ],
    )
    def k(h_hbm, e_hbm, src_hbm, dst_hbm, out_hbm,
          ebufs, hbufs, sidxs, didxs, zbuf, accum,
          sem_i, sem_e, sem_h, sem_s):
        c = lax.axis_index("c")
        s = lax.axis_index("s")
        base = c * (E // NC) + s * EPW

        def start_idx(kc, q):
            off = base + kc * ECHUNK
            pltpu.async_copy(src_hbm.at[pl.ds(off, ECHUNK)], sidxs[q],
                             sem_i[q])
            pltpu.async_copy(dst_hbm.at[pl.ds(off, ECHUNK)], didxs[q],
                             sem_i[q])

        def wait_idx(q):
            pltpu.make_async_copy(src_hbm.at[pl.ds(0, ECHUNK)], sidxs[q],
                                  sem_i[q]).wait()
            pltpu.make_async_copy(dst_hbm.at[pl.ds(0, ECHUNK)], didxs[q],
                                  sem_i[q]).wait()

        def start_e(kc, b):
            off = base + kc * ECHUNK
            pltpu.async_copy(e_hbm.at[pl.ds(off, ECHUNK)], ebufs[b],
                             sem_e[b])

        def start_gather(b, q):
            pltpu.async_copy(h_hbm.at[sidxs[q]], hbufs[b], sem_h[b])

        def wait_inputs(b, q):
            pltpu.make_async_copy(e_hbm.at[pl.ds(0, ECHUNK)], ebufs[b],
                                  sem_e[b]).wait()
            pltpu.make_async_copy(h_hbm.at[sidxs[q]], hbufs[b],
                                  sem_h[b]).wait()

        def compute(b):
            eb = ebufs[b]
            hb = hbufs[b]

            def erow(i, carry):
                for r in range(H // 16):
                    sl = pl.ds(r * 16, 16)
                    eb[i, sl] = jnp.maximum(hb[i, sl] + eb[i, sl], 0.0)
                return carry

            lax.fori_loop(0, ECHUNK, erow, 0)

        def start_scatter(b, q):
            pltpu.async_copy(ebufs[b], accum.at[didxs[q]], sem_s[b],
                             add=True)

        def wait_scatter(b):
            pltpu.make_async_copy(ebufs[b], accum.at[didxs[0]],
                                  sem_s[b]).wait()

        start_idx(0, 0)
        start_idx(1, 1)
        start_idx(2, 2)
        wait_idx(0)
        start_e(0, 0)
        start_gather(0, 0)
        wait_idx(1)
        start_e(1, 1)
        start_gather(1, 1)

        zero16 = jnp.zeros((16,), F32)

        def zrow(i, carry):
            for r in range(H // 16):
                zbuf[i, pl.ds(r * 16, 16)] = zero16
            return carry

        lax.fori_loop(0, NZROW, zrow, 0)
        row0 = s * NPT
        zcps = [pltpu.async_copy(
            zbuf, accum.at[pl.ds(row0 + kk * NZROW, NZROW)], sem_s[0])
            for kk in range(NPT // NZROW)]
        for cp in zcps:
            cp.wait()
        plsc.subcore_barrier()

        def body(kc, b, q, first):
            wait_inputs(b, q)
            compute(b)
            start_scatter(b, q)

            b2 = (b + 2) % 3
            q2 = (q + 2) % 4

            @pl.when(kc + 2 < NCH)
            def _():
                if not first:
                    wait_scatter(b2)
                start_e(kc + 2, b2)
                wait_idx(q2)
                start_gather(b2, q2)

            @pl.when(kc + 3 < NCH)
            def _():
                start_idx(kc + 3, (q + 3) % 4)

        body(0, 0, 0, True)

        def twelve(i, carry):
            for j in range(1, 13):
                kc = 12 * i + j
                body(kc, j % 3, j % 4, False)
            return carry

        nt = (NCH - 1) // 12
        lax.fori_loop(0, nt, twelve, 0)
        for kc in range(1 + 12 * nt, NCH):
            body(kc, kc % 3, kc % 4, False)

        wait_scatter(0)
        wait_scatter(1)
        wait_scatter(2)
        plsc.subcore_barrier()
        for kk in range(NPT // NZROW):
            r0 = row0 + kk * NZROW
            pltpu.sync_copy(accum.at[pl.ds(r0, NZROW)], zbuf)
            pltpu.sync_copy(zbuf, out_hbm.at[c, pl.ds(r0, NZROW)])

    return k(h, e_l, src, dst)


def _gin_update_body(h_ref, a_ref, ep_ref, w1_ref, b1_ref, w2_ref, b2_ref,
                     g_ref, be_ref, out_ref):
    h = h_ref[...]
    a = a_ref[0, :N] + a_ref[1, :N]
    h2 = h * ep_ref[...] + a
    t = jnp.maximum(
        jnp.dot(h2, w1_ref[...], preferred_element_type=F32) + b1_ref[...], 0.0)
    h2 = jnp.dot(t, w2_ref[...], preferred_element_type=F32) + b2_ref[...]
    h2 = jnp.maximum(h2, 0.0)
    mu = jnp.mean(h2, axis=0, keepdims=True)
    xc = h2 - mu
    var = jnp.mean(xc * xc, axis=0, keepdims=True)
    out_ref[...] = xc * lax.rsqrt(var + 1e-5) * g_ref[...] + be_ref[...]


def _gin_update(h, part, ep, w1, b1, w2, b2, gamma, beta):
    return pl.pallas_call(
        _gin_update_body,
        out_shape=jax.ShapeDtypeStruct((N, H), F32),
    )(h, part, ep, w1, b1, w2, b2, gamma, beta)


def _pool_sc(rows, idx):
    mesh = plsc.VectorSubcoreMesh(core_axis_name="c", subcore_axis_name="s")

    @functools.partial(
        pl.kernel,
        out_type=jax.ShapeDtypeStruct((NC, NB, GPAD, H), F32),
        mesh=mesh,
        scratch_types=[
            pltpu.VMEM((NB, PCHUNK, H), F32),
            pltpu.VMEM((PCHUNK,), jnp.int32),
            pltpu.VMEM((GROWS, H), F32),
            [pltpu.VMEM_SHARED((GPAD, H), F32) for _ in range(NB)],
            pltpu.SemaphoreType.DMA,
        ],
    )
    def k(rows_hbm, idx_hbm, out_hbm, rbuf, ridx, zbuf, accums, sem):
        c = lax.axis_index("c")
        s = lax.axis_index("s")

        zero16 = jnp.zeros((16,), F32)

        def zrow(i, carry):
            for r in range(H // 16):
                zbuf[i, pl.ds(r * 16, 16)] = zero16
            return carry

        lax.fori_loop(0, GROWS, zrow, 0)
        off0 = s * GROWS
        for b in range(NB):
            pltpu.sync_copy(zbuf, accums[b].at[pl.ds(off0, GROWS)])
        plsc.subcore_barrier()

        base = c * (NP // NC) + s * RPW

        def chunk(kc, carry):
            off = base + kc * PCHUNK
            pltpu.sync_copy(idx_hbm.at[pl.ds(off, PCHUNK)], ridx)
            cps = [pltpu.async_copy(rows_hbm.at[b, pl.ds(off, PCHUNK)],
                                    rbuf.at[b], sem) for b in range(NB)]
            for cp in cps:
                cp.wait()
            for b in range(NB):
                pltpu.sync_copy(rbuf.at[b], accums[b].at[ridx], add=True)
            return carry

        lax.fori_loop(0, RPW // PCHUNK, chunk, 0)

        plsc.subcore_barrier()
        for b in range(NB):
            pltpu.sync_copy(accums[b].at[pl.ds(off0, GROWS)], zbuf)
            pltpu.sync_copy(zbuf, out_hbm.at[c, b, pl.ds(off0, GROWS)])

    return k(rows, idx)


def _head_body(p_ref, w_ref, sg_ref, f0w, f0b, f1w, f1b, f2w, f2b, pw, pb,
               out_ref):
    q = p_ref[0] + p_ref[1]
    hsum = jnp.concatenate([q[0, :G], q[1, :G], q[2, :G]], axis=1)
    cnt = q[3, :G, 0:1]
    gmean = hsum / jnp.maximum(cnt, 1.0)
    g = jnp.maximum(
        jnp.dot(gmean, f0w[...], preferred_element_type=F32) + f0b[...], 0.0)
    sg = sg_ref[...]
    onehot = (lax.broadcasted_iota(jnp.int32, (SG, G), 0) == sg[None, :])
    ow = onehot.astype(F32) * w_ref[...]
    norm = jnp.sum(ow, axis=1, keepdims=True)
    s2 = jnp.dot(ow, g, preferred_element_type=F32) / norm
    s2 = jnp.maximum(
        jnp.dot(s2, f1w[...], preferred_element_type=F32) + f1b[...], 0.0)
    s2 = jnp.maximum(
        jnp.dot(s2, f2w[...], preferred_element_type=F32) + f2b[...], 0.0)
    out_ref[...] = jnp.dot(s2, pw[...], preferred_element_type=F32) + pb[...]


def _head(pooled, wrow, sgb, fc0, fc1, fc2, pred):
    return pl.pallas_call(
        _head_body,
        out_shape=jax.ShapeDtypeStruct((SG, OUT), F32),
    )(pooled, wrow, sgb, fc0[0], fc0[1], fc1[0], fc1[1], fc2[0], fc2[1],
      pred[0], pred[1])


def kernel(x, edge_index, edge_attr, batch, weights, subgraph_batch, params):
    convs = params["convs"]
    bns = params["bns"]

    e_all = [_edge_mlp_1(edge_attr, cp["bw1"], cp["bb1"], cp["bw2"],
                         cp["bb2"]) for cp in convs]

    src = edge_index[0]
    dst = edge_index[1]

    h = x
    hs = []
    for l in range(NL):
        part = _msg_sc(h, e_all[l], src, dst)
        cp = convs[l]
        bp = bns[l]
        ep = (1.0 + cp["eps"]).reshape(1, 1)
        h = _gin_update(h, part, ep, cp["mw1"], cp["mb1"], cp["mw2"],
                        cp["mb2"], bp["gamma"], bp["beta"])
        hs.append(h)

    rowpad = jnp.zeros((NP - N, H), F32)
    rows4 = jnp.stack([
        jnp.concatenate([hs[0], rowpad], axis=0),
        jnp.concatenate([hs[1], rowpad], axis=0),
        jnp.concatenate([hs[2], rowpad], axis=0),
        jnp.concatenate([jnp.ones((N, H), F32), rowpad], axis=0),
    ])
    bidx = jnp.concatenate([batch, jnp.zeros((NP - N,), jnp.int32)])

    pooled = _pool_sc(rows4, bidx)

    wrow = weights.reshape(1, G)
    return _head(pooled, wrow, subgraph_batch, params["fc0"], params["fc1"],
                 params["fc2"], params["pred"])


# speedup vs baseline: 1.0568x; 1.0103x over previous
import functools

import jax
import jax.numpy as jnp
import numpy as np
from jax import lax
from jax.experimental import pallas as pl
from jax.experimental.pallas import tpu as pltpu
from jax.experimental.pallas import tpu_sc as plsc

N = 10000
E = 320000
DE = 16
H = 128
NL = 3
G = 1000
SG = 100
OUT = 64

NC = 2
NS = 16
NW = NC * NS

EPW = E // NW
ECHUNK = 40
NPAD = 10240
NPT = NPAD // NS
NZROW = 64

NP = 10240
NB = 4
GPAD = 1024
RPW = NP // NW
PCHUNK = 64
GROWS = GPAD // NS

F32 = jnp.float32


BE = 8000


def _edge_mlp_body(ea_ref, w1_ref, b1_ref, w2_ref, b2_ref, out_ref):
    t = jnp.dot(ea_ref[...].astype(jnp.bfloat16),
                w1_ref[0].astype(jnp.bfloat16), preferred_element_type=F32)
    t = jnp.maximum(t + b1_ref[0, 0], 0.0)
    e = jnp.dot(t.astype(jnp.bfloat16), w2_ref[0].astype(jnp.bfloat16),
                preferred_element_type=F32)
    out_ref[...] = e + b2_ref[0, 0]


def _edge_mlp_1(edge_attr, w1, b1, w2, b2):
    return pl.pallas_call(
        _edge_mlp_body,
        grid=(E // BE,),
        in_specs=[
            pl.BlockSpec((BE, DE), lambda i: (i, 0)),
            pl.BlockSpec((1, DE, H), lambda i: (0, 0, 0)),
            pl.BlockSpec((1, 1, H), lambda i: (0, 0, 0)),
            pl.BlockSpec((1, H, H), lambda i: (0, 0, 0)),
            pl.BlockSpec((1, 1, H), lambda i: (0, 0, 0)),
        ],
        out_specs=pl.BlockSpec((BE, H), lambda i: (i, 0)),
        out_shape=jax.ShapeDtypeStruct((E, H), F32),
    )(edge_attr, w1[None], b1.reshape(1, 1, H), w2[None],
      b2.reshape(1, 1, H))


NCH = EPW // ECHUNK


def _msg_sc(h, e_l, src, dst):
    mesh = plsc.VectorSubcoreMesh(core_axis_name="c", subcore_axis_name="s")

    @functools.partial(
        pl.kernel,
        out_type=jax.ShapeDtypeStruct((NC, NPAD, H), F32),
        mesh=mesh,
        scratch_types=[
            [pltpu.VMEM((ECHUNK, H), F32) for _ in range(3)],
            [pltpu.VMEM((ECHUNK, H), F32) for _ in range(3)],
            [pltpu.VMEM((ECHUNK,), jnp.int32) for _ in range(4)],
            [pltpu.VMEM((ECHUNK,), jnp.int32) for _ in range(4)],
            pltpu.VMEM((NZROW, H), F32),
            pltpu.VMEM_SHARED((NPAD, H), F32),
            [pltpu.SemaphoreType.DMA for _ in range(4)],
            [pltpu.SemaphoreType.DMA for _ in range(3)],
            [pltpu.SemaphoreType.DMA for _ in range(3)],
            [pltpu.SemaphoreType.DMA for _ in range(3)],
        ],
    )
    def k(h_hbm, e_hbm, src_hbm, dst_hbm, out_hbm,
          ebufs, hbufs, sidxs, didxs, zbuf, accum,
          sem_i, sem_e, sem_h, sem_s):
        c = lax.axis_index("c")
        s = lax.axis_index("s")
        base = c * (E // NC) + s * EPW

        def start_idx(kc, q):
            off = base + kc * ECHUNK
            pltpu.async_copy(src_hbm.at[pl.ds(off, ECHUNK)], sidxs[q],
                             sem_i[q])
            pltpu.async_copy(dst_hbm.at[pl.ds(off, ECHUNK)], didxs[q],
                             sem_i[q])

        def wait_idx(q):
            pltpu.make_async_copy(src_hbm.at[pl.ds(0, ECHUNK)], sidxs[q],
                                  sem_i[q]).wait()
            pltpu.make_async_copy(dst_hbm.at[pl.ds(0, ECHUNK)], didxs[q],
                                  sem_i[q]).wait()

        def start_e(kc, b):
            off = base + kc * ECHUNK
            pltpu.async_copy(e_hbm.at[pl.ds(off, ECHUNK)], ebufs[b],
                             sem_e[b])

        def start_gather(b, q):
            pltpu.async_copy(h_hbm.at[sidxs[q]], hbufs[b], sem_h[b])

        def wait_inputs(b, q):
            pltpu.make_async_copy(e_hbm.at[pl.ds(0, ECHUNK)], ebufs[b],
                                  sem_e[b]).wait()
            pltpu.make_async_copy(h_hbm.at[sidxs[q]], hbufs[b],
                                  sem_h[b]).wait()

        def compute(b):
            eb = ebufs[b]
            hb = hbufs[b]

            def erow(i, carry):
                for r in range(H // 16):
                    sl = pl.ds(r * 16, 16)
                    eb[i, sl] = jnp.maximum(hb[i, sl] + eb[i, sl], 0.0)
                return carry

            lax.fori_loop(0, ECHUNK, erow, 0)

        def start_scatter(b, q):
            pltpu.async_copy(ebufs[b], accum.at[didxs[q]], sem_s[b],
                             add=True)

        def wait_scatter(b):
            pltpu.make_async_copy(ebufs[b], accum.at[didxs[0]],
                                  sem_s[b]).wait()

        start_idx(0, 0)
        start_idx(1, 1)
        start_idx(2, 2)
        wait_idx(0)
        start_e(0, 0)
        start_gather(0, 0)
        wait_idx(1)
        start_e(1, 1)
        start_gather(1, 1)

        zero16 = jnp.zeros((16,), F32)

        def zrow(i, carry):
            for r in range(H // 16):
                zbuf[i, pl.ds(r * 16, 16)] = zero16
            return carry

        lax.fori_loop(0, NZROW, zrow, 0)
        row0 = s * NPT
        zcps = [pltpu.async_copy(
            zbuf, accum.at[pl.ds(row0 + kk * NZROW, NZROW)], sem_s[0])
            for kk in range(NPT // NZROW)]
        for cp in zcps:
            cp.wait()
        plsc.subcore_barrier()

        def body(kc, b, q, first):
            wait_inputs(b, q)
            compute(b)
            start_scatter(b, q)

            b2 = (b + 2) % 3
            q2 = (q + 2) % 4

            @pl.when(kc + 2 < NCH)
            def _():
                if not first:
                    wait_scatter(b2)
                start_e(kc + 2, b2)
                wait_idx(q2)
                start_gather(b2, q2)

            @pl.when(kc + 3 < NCH)
            def _():
                start_idx(kc + 3, (q + 3) % 4)

        body(0, 0, 0, True)

        def twelve(i, carry):
            for j in range(1, 13):
                kc = 12 * i + j
                body(kc, j % 3, j % 4, False)
            return carry

        nt = (NCH - 1) // 12
        lax.fori_loop(0, nt, twelve, 0)
        for kc in range(1 + 12 * nt, NCH):
            body(kc, kc % 3, kc % 4, False)

        wait_scatter(0)
        wait_scatter(1)
        wait_scatter(2)
        plsc.subcore_barrier()
        for kk in range(NPT // NZROW):
            r0 = row0 + kk * NZROW
            pltpu.sync_copy(accum.at[pl.ds(r0, NZROW)], zbuf)
            pltpu.sync_copy(zbuf, out_hbm.at[c, pl.ds(r0, NZROW)])

    return k(h, e_l, src, dst)


def _gin_update_body(h_ref, a_ref, ep_ref, w1_ref, b1_ref, w2_ref, b2_ref,
                     g_ref, be_ref, out_ref):
    h = h_ref[...]
    a = a_ref[0, :N] + a_ref[1, :N]
    h2 = h * ep_ref[...] + a
    t = jnp.maximum(
        jnp.dot(h2, w1_ref[...], preferred_element_type=F32) + b1_ref[...], 0.0)
    h2 = jnp.dot(t, w2_ref[...], preferred_element_type=F32) + b2_ref[...]
    h2 = jnp.maximum(h2, 0.0)
    mu = jnp.mean(h2, axis=0, keepdims=True)
    xc = h2 - mu
    var = jnp.mean(xc * xc, axis=0, keepdims=True)
    out_ref[...] = xc * lax.rsqrt(var + 1e-5) * g_ref[...] + be_ref[...]


def _gin_update(h, part, ep, w1, b1, w2, b2, gamma, beta):
    return pl.pallas_call(
        _gin_update_body,
        out_shape=jax.ShapeDtypeStruct((N, H), F32),
    )(h, part, ep, w1, b1, w2, b2, gamma, beta)


def _pool_sc(rows, idx):
    mesh = plsc.VectorSubcoreMesh(core_axis_name="c", subcore_axis_name="s")

    @functools.partial(
        pl.kernel,
        out_type=jax.ShapeDtypeStruct((NC, NB, GPAD, H), F32),
        mesh=mesh,
        scratch_types=[
            pltpu.VMEM((NB, PCHUNK, H), F32),
            pltpu.VMEM((PCHUNK,), jnp.int32),
            pltpu.VMEM((GROWS, H), F32),
            [pltpu.VMEM_SHARED((GPAD, H), F32) for _ in range(NB)],
            pltpu.SemaphoreType.DMA,
        ],
    )
    def k(rows_hbm, idx_hbm, out_hbm, rbuf, ridx, zbuf, accums, sem):
        c = lax.axis_index("c")
        s = lax.axis_index("s")

        zero16 = jnp.zeros((16,), F32)

        def zrow(i, carry):
            for r in range(H // 16):
                zbuf[i, pl.ds(r * 16, 16)] = zero16
            return carry

        lax.fori_loop(0, GROWS, zrow, 0)
        off0 = s * GROWS
        for b in range(NB):
            pltpu.sync_copy(zbuf, accums[b].at[pl.ds(off0, GROWS)])
        plsc.subcore_barrier()

        base = c * (NP // NC) + s * RPW

        def chunk(kc, carry):
            off = base + kc * PCHUNK
            pltpu.sync_copy(idx_hbm.at[pl.ds(off, PCHUNK)], ridx)
            cps = [pltpu.async_copy(rows_hbm.at[b, pl.ds(off, PCHUNK)],
                                    rbuf.at[b], sem) for b in range(NB)]
            for cp in cps:
                cp.wait()
            for b in range(NB):
                pltpu.sync_copy(rbuf.at[b], accums[b].at[ridx], add=True)
            return carry

        lax.fori_loop(0, RPW // PCHUNK, chunk, 0)

        plsc.subcore_barrier()
        for b in range(NB):
            pltpu.sync_copy(accums[b].at[pl.ds(off0, GROWS)], zbuf)
            pltpu.sync_copy(zbuf, out_hbm.at[c, b, pl.ds(off0, GROWS)])

    return k(rows, idx)


def _head_body(p_ref, w_ref, sg_ref, f0w, f0b, f1w, f1b, f2w, f2b, pw, pb,
               out_ref):
    q = p_ref[0] + p_ref[1]
    hsum = jnp.concatenate([q[0, :G], q[1, :G], q[2, :G]], axis=1)
    cnt = q[3, :G, 0:1]
    gmean = hsum / jnp.maximum(cnt, 1.0)
    g = jnp.maximum(
        jnp.dot(gmean, f0w[...], preferred_element_type=F32) + f0b[...], 0.0)
    sg = sg_ref[...]
    onehot = (lax.broadcasted_iota(jnp.int32, (SG, G), 0) == sg[None, :])
    ow = onehot.astype(F32) * w_ref[...]
    norm = jnp.sum(ow, axis=1, keepdims=True)
    s2 = jnp.dot(ow, g, preferred_element_type=F32) / norm
    s2 = jnp.maximum(
        jnp.dot(s2, f1w[...], preferred_element_type=F32) + f1b[...], 0.0)
    s2 = jnp.maximum(
        jnp.dot(s2, f2w[...], preferred_element_type=F32) + f2b[...], 0.0)
    out_ref[...] = jnp.dot(s2, pw[...], preferred_element_type=F32) + pb[...]


def _head(pooled, wrow, sgb, fc0, fc1, fc2, pred):
    return pl.pallas_call(
        _head_body,
        out_shape=jax.ShapeDtypeStruct((SG, OUT), F32),
    )(pooled, wrow, sgb, fc0[0], fc0[1], fc1[0], fc1[1], fc2[0], fc2[1],
      pred[0], pred[1])


def kernel(x, edge_index, edge_attr, batch, weights, subgraph_batch, params):
    convs = params["convs"]
    bns = params["bns"]

    e_all = [_edge_mlp_1(edge_attr, cp["bw1"], cp["bb1"], cp["bw2"],
                         cp["bb2"]) for cp in convs]

    src = edge_index[0]
    dst = edge_index[1]

    h = x
    hs = []
    for l in range(NL):
        part = _msg_sc(h, e_all[l], src, dst)
        cp = convs[l]
        bp = bns[l]
        ep = (1.0 + cp["eps"]).reshape(1, 1)
        h = _gin_update(h, part, ep, cp["mw1"], cp["mb1"], cp["mw2"],
                        cp["mb2"], bp["gamma"], bp["beta"])
        hs.append(h)

    rowpad = jnp.zeros((NP - N, H), F32)
    rows4 = jnp.stack([
        jnp.concatenate([hs[0], rowpad], axis=0),
        jnp.concatenate([hs[1], rowpad], axis=0),
        jnp.concatenate([hs[2], rowpad], axis=0),
        jnp.concatenate([jnp.ones((N, H), F32), rowpad], axis=0),
    ])
    bidx = jnp.concatenate([batch, jnp.zeros((NP - N,), jnp.int32)])

    pooled = _pool_sc(rows4, bidx)

    wrow = weights.reshape(1, G)
    return _head(pooled, wrow, subgraph_batch, params["fc0"], params["fc1"],
                 params["fc2"], params["pred"])


# speedup vs baseline: 1.0577x; 1.0009x over previous
import functools

import jax
import jax.numpy as jnp
import numpy as np
from jax import lax
from jax.experimental import pallas as pl
from jax.experimental.pallas import tpu as pltpu
from jax.experimental.pallas import tpu_sc as plsc

N = 10000
E = 320000
DE = 16
H = 128
NL = 3
G = 1000
SG = 100
OUT = 64

NC = 2
NS = 16
NW = NC * NS

EPW = E // NW
ECHUNK = 40
NPAD = 10240
NPT = NPAD // NS
NZROW = 64

NP = 10240
NB = 4
GPAD = 1024
RPW = NP // NW
PCHUNK = 64
GROWS = GPAD // NS

F32 = jnp.float32


BE = 16000


def _edge_mlp_body(ea_ref, w1_ref, b1_ref, w2_ref, b2_ref, out_ref):
    t = jnp.dot(ea_ref[...].astype(jnp.bfloat16),
                w1_ref[0].astype(jnp.bfloat16), preferred_element_type=F32)
    t = jnp.maximum(t + b1_ref[0, 0], 0.0)
    e = jnp.dot(t.astype(jnp.bfloat16), w2_ref[0].astype(jnp.bfloat16),
                preferred_element_type=F32)
    out_ref[...] = e + b2_ref[0, 0]


def _edge_mlp_1(edge_attr, w1, b1, w2, b2):
    return pl.pallas_call(
        _edge_mlp_body,
        grid=(E // BE,),
        in_specs=[
            pl.BlockSpec((BE, DE), lambda i: (i, 0)),
            pl.BlockSpec((1, DE, H), lambda i: (0, 0, 0)),
            pl.BlockSpec((1, 1, H), lambda i: (0, 0, 0)),
            pl.BlockSpec((1, H, H), lambda i: (0, 0, 0)),
            pl.BlockSpec((1, 1, H), lambda i: (0, 0, 0)),
        ],
        out_specs=pl.BlockSpec((BE, H), lambda i: (i, 0)),
        out_shape=jax.ShapeDtypeStruct((E, H), F32),
    )(edge_attr, w1[None], b1.reshape(1, 1, H), w2[None],
      b2.reshape(1, 1, H))


NCH = EPW // ECHUNK


def _msg_sc(h, e_l, src, dst):
    mesh = plsc.VectorSubcoreMesh(core_axis_name="c", subcore_axis_name="s")

    @functools.partial(
        pl.kernel,
        out_type=jax.ShapeDtypeStruct((NC, NPAD, H), F32),
        mesh=mesh,
        scratch_types=[
            [pltpu.VMEM((ECHUNK, H), F32) for _ in range(3)],
            [pltpu.VMEM((ECHUNK, H), F32) for _ in range(3)],
            [pltpu.VMEM((ECHUNK,), jnp.int32) for _ in range(4)],
            [pltpu.VMEM((ECHUNK,), jnp.int32) for _ in range(4)],
            pltpu.VMEM((NZROW, H), F32),
            pltpu.VMEM_SHARED((NPAD, H), F32),
            [pltpu.SemaphoreType.DMA for _ in range(4)],
            [pltpu.SemaphoreType.DMA for _ in range(3)],
            [pltpu.SemaphoreType.DMA for _ in range(3)],
            [pltpu.SemaphoreType.DMA for _ in range(3)],
        ],
    )
    def k(h_hbm, e_hbm, src_hbm, dst_hbm, out_hbm,
          ebufs, hbufs, sidxs, didxs, zbuf, accum,
          sem_i, sem_e, sem_h, sem_s):
        c = lax.axis_index("c")
        s = lax.axis_index("s")
        base = c * (E // NC) + s * EPW

        def start_idx(kc, q):
            off = base + kc * ECHUNK
            pltpu.async_copy(src_hbm.at[pl.ds(off, ECHUNK)], sidxs[q],
                             sem_i[q])
            pltpu.async_copy(dst_hbm.at[pl.ds(off, ECHUNK)], didxs[q],
                             sem_i[q])

        def wait_idx(q):
            pltpu.make_async_copy(src_hbm.at[pl.ds(0, ECHUNK)], sidxs[q],
                                  sem_i[q]).wait()
            pltpu.make_async_copy(dst_hbm.at[pl.ds(0, ECHUNK)], didxs[q],
                                  sem_i[q]).wait()

        def start_e(kc, b):
            off = base + kc * ECHUNK
            pltpu.async_copy(e_hbm.at[pl.ds(off, ECHUNK)], ebufs[b],
                             sem_e[b])

        def start_gather(b, q):
            pltpu.async_copy(h_hbm.at[sidxs[q]], hbufs[b], sem_h[b])

        def wait_inputs(b, q):
            pltpu.make_async_copy(e_hbm.at[pl.ds(0, ECHUNK)], ebufs[b],
                                  sem_e[b]).wait()
            pltpu.make_async_copy(h_hbm.at[sidxs[q]], hbufs[b],
                                  sem_h[b]).wait()

        def compute(b):
            eb = ebufs[b]
            hb = hbufs[b]

            def erow(i, carry):
                for r in range(H // 16):
                    sl = pl.ds(r * 16, 16)
                    eb[i, sl] = jnp.maximum(hb[i, sl] + eb[i, sl], 0.0)
                return carry

            lax.fori_loop(0, ECHUNK, erow, 0)

        def start_scatter(b, q):
            pltpu.async_copy(ebufs[b], accum.at[didxs[q]], sem_s[b],
                             add=True)

        def wait_scatter(b):
            pltpu.make_async_copy(ebufs[b], accum.at[didxs[0]],
                                  sem_s[b]).wait()

        start_idx(0, 0)
        start_idx(1, 1)
        start_idx(2, 2)
        wait_idx(0)
        start_e(0, 0)
        start_gather(0, 0)
        wait_idx(1)
        start_e(1, 1)
        start_gather(1, 1)

        zero16 = jnp.zeros((16,), F32)

        def zrow(i, carry):
            for r in range(H // 16):
                zbuf[i, pl.ds(r * 16, 16)] = zero16
            return carry

        lax.fori_loop(0, NZROW, zrow, 0)
        row0 = s * NPT
        zcps = [pltpu.async_copy(
            zbuf, accum.at[pl.ds(row0 + kk * NZROW, NZROW)], sem_s[0])
            for kk in range(NPT // NZROW)]
        for cp in zcps:
            cp.wait()
        plsc.subcore_barrier()

        def body(kc, b, q, first):
            wait_inputs(b, q)
            compute(b)
            start_scatter(b, q)

            b2 = (b + 2) % 3
            q2 = (q + 2) % 4

            @pl.when(kc + 2 < NCH)
            def _():
                if not first:
                    wait_scatter(b2)
                start_e(kc + 2, b2)
                wait_idx(q2)
                start_gather(b2, q2)

            @pl.when(kc + 3 < NCH)
            def _():
                start_idx(kc + 3, (q + 3) % 4)

        body(0, 0, 0, True)

        def twelve(i, carry):
            for j in range(1, 13):
                kc = 12 * i + j
                body(kc, j % 3, j % 4, False)
            return carry

        nt = (NCH - 1) // 12
        lax.fori_loop(0, nt, twelve, 0)
        for kc in range(1 + 12 * nt, NCH):
            body(kc, kc % 3, kc % 4, False)

        wait_scatter(0)
        wait_scatter(1)
        wait_scatter(2)
        plsc.subcore_barrier()
        for kk in range(NPT // NZROW):
            r0 = row0 + kk * NZROW
            pltpu.sync_copy(accum.at[pl.ds(r0, NZROW)], zbuf)
            pltpu.sync_copy(zbuf, out_hbm.at[c, pl.ds(r0, NZROW)])

    return k(h, e_l, src, dst)


def _gin_update_body(h_ref, a_ref, ep_ref, w1_ref, b1_ref, w2_ref, b2_ref,
                     g_ref, be_ref, out_ref):
    h = h_ref[...]
    a = a_ref[0, :N] + a_ref[1, :N]
    h2 = h * ep_ref[...] + a
    t = jnp.maximum(
        jnp.dot(h2, w1_ref[...], preferred_element_type=F32) + b1_ref[...], 0.0)
    h2 = jnp.dot(t, w2_ref[...], preferred_element_type=F32) + b2_ref[...]
    h2 = jnp.maximum(h2, 0.0)
    mu = jnp.mean(h2, axis=0, keepdims=True)
    xc = h2 - mu
    var = jnp.mean(xc * xc, axis=0, keepdims=True)
    out_ref[...] = xc * lax.rsqrt(var + 1e-5) * g_ref[...] + be_ref[...]


def _gin_update(h, part, ep, w1, b1, w2, b2, gamma, beta):
    return pl.pallas_call(
        _gin_update_body,
        out_shape=jax.ShapeDtypeStruct((N, H), F32),
    )(h, part, ep, w1, b1, w2, b2, gamma, beta)


def _pool_sc(rows, idx):
    mesh = plsc.VectorSubcoreMesh(core_axis_name="c", subcore_axis_name="s")

    @functools.partial(
        pl.kernel,
        out_type=jax.ShapeDtypeStruct((NC, NB, GPAD, H), F32),
        mesh=mesh,
        scratch_types=[
            pltpu.VMEM((NB, PCHUNK, H), F32),
            pltpu.VMEM((PCHUNK,), jnp.int32),
            pltpu.VMEM((GROWS, H), F32),
            [pltpu.VMEM_SHARED((GPAD, H), F32) for _ in range(NB)],
            pltpu.SemaphoreType.DMA,
        ],
    )
    def k(rows_hbm, idx_hbm, out_hbm, rbuf, ridx, zbuf, accums, sem):
        c = lax.axis_index("c")
        s = lax.axis_index("s")

        zero16 = jnp.zeros((16,), F32)

        def zrow(i, carry):
            for r in range(H // 16):
                zbuf[i, pl.ds(r * 16, 16)] = zero16
            return carry

        lax.fori_loop(0, GROWS, zrow, 0)
        off0 = s * GROWS
        for b in range(NB):
            pltpu.sync_copy(zbuf, accums[b].at[pl.ds(off0, GROWS)])
        plsc.subcore_barrier()

        base = c * (NP // NC) + s * RPW

        def chunk(kc, carry):
            off = base + kc * PCHUNK
            pltpu.sync_copy(idx_hbm.at[pl.ds(off, PCHUNK)], ridx)
            cps = [pltpu.async_copy(rows_hbm.at[b, pl.ds(off, PCHUNK)],
                                    rbuf.at[b], sem) for b in range(NB)]
            for cp in cps:
                cp.wait()
            for b in range(NB):
                pltpu.sync_copy(rbuf.at[b], accums[b].at[ridx], add=True)
            return carry

        lax.fori_loop(0, RPW // PCHUNK, chunk, 0)

        plsc.subcore_barrier()
        for b in range(NB):
            pltpu.sync_copy(accums[b].at[pl.ds(off0, GROWS)], zbuf)
            pltpu.sync_copy(zbuf, out_hbm.at[c, b, pl.ds(off0, GROWS)])

    return k(rows, idx)


def _head_body(p_ref, w_ref, sg_ref, f0w, f0b, f1w, f1b, f2w, f2b, pw, pb,
               out_ref):
    q = p_ref[0] + p_ref[1]
    hsum = jnp.concatenate([q[0, :G], q[1, :G], q[2, :G]], axis=1)
    cnt = q[3, :G, 0:1]
    gmean = hsum / jnp.maximum(cnt, 1.0)
    g = jnp.maximum(
        jnp.dot(gmean, f0w[...], preferred_element_type=F32) + f0b[...], 0.0)
    sg = sg_ref[...]
    onehot = (lax.broadcasted_iota(jnp.int32, (SG, G), 0) == sg[None, :])
    ow = onehot.astype(F32) * w_ref[...]
    norm = jnp.sum(ow, axis=1, keepdims=True)
    s2 = jnp.dot(ow, g, preferred_element_type=F32) / norm
    s2 = jnp.maximum(
        jnp.dot(s2, f1w[...], preferred_element_type=F32) + f1b[...], 0.0)
    s2 = jnp.maximum(
        jnp.dot(s2, f2w[...], preferred_element_type=F32) + f2b[...], 0.0)
    out_ref[...] = jnp.dot(s2, pw[...], preferred_element_type=F32) + pb[...]


def _head(pooled, wrow, sgb, fc0, fc1, fc2, pred):
    return pl.pallas_call(
        _head_body,
        out_shape=jax.ShapeDtypeStruct((SG, OUT), F32),
    )(pooled, wrow, sgb, fc0[0], fc0[1], fc1[0], fc1[1], fc2[0], fc2[1],
      pred[0], pred[1])


def kernel(x, edge_index, edge_attr, batch, weights, subgraph_batch, params):
    convs = params["convs"]
    bns = params["bns"]

    e_all = [_edge_mlp_1(edge_attr, cp["bw1"], cp["bb1"], cp["bw2"],
                         cp["bb2"]) for cp in convs]

    src = edge_index[0]
    dst = edge_index[1]

    h = x
    hs = []
    for l in range(NL):
        part = _msg_sc(h, e_all[l], src, dst)
        cp = convs[l]
        bp = bns[l]
        ep = (1.0 + cp["eps"]).reshape(1, 1)
        h = _gin_update(h, part, ep, cp["mw1"], cp["mb1"], cp["mw2"],
                        cp["mb2"], bp["gamma"], bp["beta"])
        hs.append(h)

    rowpad = jnp.zeros((NP - N, H), F32)
    rows4 = jnp.stack([
        jnp.concatenate([hs[0], rowpad], axis=0),
        jnp.concatenate([hs[1], rowpad], axis=0),
        jnp.concatenate([hs[2], rowpad], axis=0),
        jnp.concatenate([jnp.ones((N, H), F32), rowpad], axis=0),
    ])
    bidx = jnp.concatenate([batch, jnp.zeros((NP - N,), jnp.int32)])

    pooled = _pool_sc(rows4, bidx)

    wrow = weights.reshape(1, G)
    return _head(pooled, wrow, subgraph_batch, params["fc0"], params["fc1"],
                 params["fc2"], params["pred"])
